# Initial kernel scaffold; baseline (speedup 1.0000x reference)
import jax
import jax.numpy as jnp
from jax.experimental import pallas as pl


def kernel(xs, B):
    raise NotImplementedError("write your pallas kernel here")


# speedup vs baseline: 5.4824x; 5.4824x over previous
import functools

import numpy as np
import jax
import jax.numpy as jnp
from jax.experimental import pallas as pl
from jax.experimental.pallas import tpu as pltpu

H = 0.001
Q = 3
N_COLS = 1024
BR = 512


def _tc_body(xs_ref, b_ref, out_ref):
    x = xs_ref[...]
    fi = (x / H).astype(jnp.int32)
    xm = x - fi.astype(jnp.float32) * H
    col = jax.lax.broadcasted_iota(jnp.int32, (BR, N_COLS), 1)
    delta = col - fi
    acc = jnp.zeros((BR, N_COLS), jnp.float32)
    for j in range(Q + 1):
        xe = xm + np.float32(H) * np.float32(Q - j)
        v = ((b_ref[j, 3] * xe + b_ref[j, 2]) * xe + b_ref[j, 1]) * xe + b_ref[j, 0]
        acc = jnp.where(delta == j, v, acc)
    out_ref[...] = acc


@jax.jit
def kernel(xs, B):
    n_xs = xs.shape[0]
    grid = (n_xs // BR,)
    out = pl.pallas_call(
        _tc_body,
        grid=grid,
        in_specs=[
            pl.BlockSpec((BR, 1), lambda i: (i, 0)),
            pl.BlockSpec(memory_space=pltpu.SMEM),
        ],
        out_specs=pl.BlockSpec((BR, N_COLS), lambda i: (i, 0)),
        out_shape=jax.ShapeDtypeStruct((n_xs, N_COLS), jnp.float32),
        compiler_params=pltpu.CompilerParams(
            dimension_semantics=("parallel",),
        ),
    )(xs.reshape(n_xs, 1), B)
    return out
